# baseline (speedup 1.0000x reference)
import functools

import jax
import jax.numpy as jnp
from jax import lax
from jax.experimental import pallas as pl
from jax.experimental.pallas import tpu as pltpu
from jax.experimental.pallas import tpu_sc as plsc

_N = 10000
_D = 128
_E = 320000
_NC = 2
_NS = 16
_CH = 128
_Q = 4
_NBLK = -(-_E // (_CH * _NS * _Q))
_NPW = _NBLK * _Q
_EPAD = _NPW * _CH * _NS
_RT = 640
_NA = _RT * _NS
_HPAD = 8


def _sc_two_sided_accumulate(h_pad, eidx, zero_blk):
    mesh = plsc.VectorSubcoreMesh(core_axis_name="c", subcore_axis_name="s")

    @functools.partial(
        pl.kernel,
        out_type=jax.ShapeDtypeStruct((_NC, _NA, _D), jnp.float32),
        mesh=mesh,
        scratch_types=[
            pltpu.VMEM((_Q, _CH), jnp.int32),
            pltpu.VMEM((_Q, _CH), jnp.int32),
            pltpu.VMEM((_CH, _D), jnp.float32),
            pltpu.VMEM((_CH, _D), jnp.float32),
            pltpu.VMEM_SHARED((_NA, _D), jnp.float32),
            *[pltpu.SemaphoreType.DMA for _ in range(4)],
        ],
    )
    def k(h_hbm, e_hbm, z_hbm, out_hbm, sg, sd, r0, r1, acc, *sems):
        rows = [r0, r1]
        gsem = sems[0:2]
        ssem = sems[2:4]
        c = lax.axis_index("c")
        s = lax.axis_index("s")

        pltpu.sync_copy(z_hbm, acc.at[pl.ds(s * _RT, _RT)])
        plsc.subcore_barrier()

        def body(blk, carry):
            pltpu.sync_copy(e_hbm.at[c, s, blk], sg)
            pltpu.sync_copy(e_hbm.at[1, s, blk], sd)
            for u in range(_Q):
                pltpu.async_copy(h_hbm.at[sg.at[u]], rows[u % 2],
                                 gsem[u % 2]).wait()
                pltpu.sync_copy(rows[u % 2], acc.at[sd.at[u]], add=True)
            return carry

        lax.fori_loop(0, _NBLK, body, 0)
        plsc.subcore_barrier()

        pltpu.sync_copy(acc.at[pl.ds(s * _RT, _RT)],
                        out_hbm.at[c, pl.ds(s * _RT, _RT)])

    return k(h_pad, eidx, zero_blk)


_BN = 400


def _tc_finalize(partials):
    def body(p_ref, o_ref):
        agg = p_ref[0] - p_ref[1]
        ss = jnp.sum(agg * agg, axis=1, keepdims=True)
        o_ref[...] = agg / (jnp.sqrt(ss) + 1e-7)

    return pl.pallas_call(
        body,
        grid=(_N // _BN,),
        in_specs=[pl.BlockSpec((_NC, _BN, _D), lambda i: (0, i, 0))],
        out_specs=pl.BlockSpec((_BN, _D), lambda i: (i, 0)),
        out_shape=jax.ShapeDtypeStruct((_N, _D), jnp.float32),
    )(partials)


def kernel(h, edge_index, r):
    eidx = jnp.concatenate(
        [edge_index.astype(jnp.int32),
         jnp.full((2, _EPAD - _E), _N, jnp.int32)], axis=1)
    eidx = eidx.reshape(2, _NS, _NBLK, _Q, _CH)
    h_pad = jnp.concatenate(
        [h, jnp.zeros((_HPAD, _D), jnp.float32)], axis=0)
    zero_blk = jnp.zeros((_RT, _D), jnp.float32)
    partials = _sc_two_sided_accumulate(h_pad, eidx, zero_blk)
    return _tc_finalize(partials)


# speedup vs baseline: 1.0413x; 1.0413x over previous
import functools

import jax
import jax.numpy as jnp
from jax import lax
from jax.experimental import pallas as pl
from jax.experimental.pallas import tpu as pltpu
from jax.experimental.pallas import tpu_sc as plsc

_N = 10000
_D = 128
_E = 320000
_NC = 2
_NS = 16
_CH = 128
_Q = 4
_NBLK = -(-_E // (_CH * _NS * _Q))
_NPW = _NBLK * _Q
_EPAD = _NPW * _CH * _NS
_RT = 640
_NA = _RT * _NS
_HPAD = 8


def _sc_two_sided_accumulate(h_pad, eidx, zero_blk):
    mesh = plsc.VectorSubcoreMesh(core_axis_name="c", subcore_axis_name="s")

    @functools.partial(
        pl.kernel,
        out_type=jax.ShapeDtypeStruct((_NC, _NA, _D), jnp.float32),
        mesh=mesh,
        scratch_types=[
            *[pltpu.VMEM((_CH,), jnp.int32) for _ in range(2)],
            *[pltpu.VMEM((_CH,), jnp.int32) for _ in range(2)],
            pltpu.VMEM((_CH, _D), jnp.float32),
            pltpu.VMEM((_CH, _D), jnp.float32),
            pltpu.VMEM_SHARED((_NA, _D), jnp.float32),
            *[pltpu.SemaphoreType.DMA for _ in range(4)],
        ],
    )
    def k(h_hbm, e_hbm, z_hbm, out_hbm, g0, g1, d0, d1, r0, r1, acc, *sems):
        gi = [g0, g1]
        di = [d0, d1]
        rows = [r0, r1]
        gsem = sems[0:2]
        ssem = sems[2:4]
        c = lax.axis_index("c")
        s = lax.axis_index("s")

        pltpu.sync_copy(z_hbm, acc.at[pl.ds(s * _RT, _RT)])
        plsc.subcore_barrier()

        def body(i, carry):
            cps = {}
            for u in range(2):
                off = (s * _NPW + i * 2 + u) * _CH
                pltpu.sync_copy(e_hbm.at[c, pl.ds(off, _CH)], gi[u])
                pltpu.sync_copy(e_hbm.at[1, pl.ds(off, _CH)], di[u])
                cps[u] = pltpu.async_copy(h_hbm.at[gi[u]], rows[u], gsem[u])
            scs = {}
            for u in range(2):
                cps[u].wait()
                scs[u] = pltpu.async_copy(rows[u], acc.at[di[u]], ssem[u],
                                          add=True)
            for u in range(2):
                scs[u].wait()
            return carry

        lax.fori_loop(0, _NPW // 2, body, 0)
        plsc.subcore_barrier()

        pltpu.sync_copy(acc.at[pl.ds(s * _RT, _RT)],
                        out_hbm.at[c, pl.ds(s * _RT, _RT)])

    return k(h_pad, eidx, zero_blk)


_BN = 400


def _tc_finalize(partials):
    def body(p_ref, o_ref):
        agg = p_ref[0] - p_ref[1]
        ss = jnp.sum(agg * agg, axis=1, keepdims=True)
        o_ref[...] = agg / (jnp.sqrt(ss) + 1e-7)

    return pl.pallas_call(
        body,
        grid=(_N // _BN,),
        in_specs=[pl.BlockSpec((_NC, _BN, _D), lambda i: (0, i, 0))],
        out_specs=pl.BlockSpec((_BN, _D), lambda i: (i, 0)),
        out_shape=jax.ShapeDtypeStruct((_N, _D), jnp.float32),
    )(partials)


def kernel(h, edge_index, r):
    eidx = jnp.concatenate(
        [edge_index.astype(jnp.int32),
         jnp.full((2, _EPAD - _E), _N, jnp.int32)], axis=1)
    h_pad = jnp.concatenate(
        [h, jnp.zeros((_HPAD, _D), jnp.float32)], axis=0)
    zero_blk = jnp.zeros((_RT, _D), jnp.float32)
    partials = _sc_two_sided_accumulate(h_pad, eidx, zero_blk)
    return _tc_finalize(partials)


# speedup vs baseline: 1.5597x; 1.4978x over previous
import functools

import jax
import jax.numpy as jnp
from jax import lax
from jax.experimental import pallas as pl
from jax.experimental.pallas import tpu as pltpu
from jax.experimental.pallas import tpu_sc as plsc

_N = 10000
_D = 128
_E = 320000
_NC = 2
_NS = 16
_CH = 128
_NPW = -(-_E // (_CH * _NS))
_EPAD = _NPW * _CH * _NS
_RT = 632
_NA = _RT * _NS
_HPAD = 8


def _sc_two_sided_accumulate(h_pad, eidx, zero_blk):
    mesh = plsc.VectorSubcoreMesh(core_axis_name="c", subcore_axis_name="s")

    @functools.partial(
        pl.kernel,
        out_type=jax.ShapeDtypeStruct((_NC, _NA, _D), jnp.float32),
        mesh=mesh,
        scratch_types=[
            pltpu.VMEM((_CH,), jnp.int32),
            pltpu.VMEM((_CH,), jnp.int32),
            pltpu.VMEM((_CH, _D), jnp.float32),
            pltpu.VMEM_SHARED((_NA, _D), jnp.float32),
            pltpu.SemaphoreType.DMA,
        ],
    )
    def k(h_hbm, e_hbm, z_hbm, out_hbm, gidx_v, didx_v, rows_v, acc, sem):
        c = lax.axis_index("c")
        s = lax.axis_index("s")

        pltpu.sync_copy(z_hbm, acc.at[pl.ds(s * _RT, _RT)])
        plsc.subcore_barrier()

        def body(j, carry):
            e0 = (s * _NPW + j) * _CH
            pltpu.sync_copy(e_hbm.at[c, pl.ds(e0, _CH)], gidx_v)
            pltpu.sync_copy(e_hbm.at[1, pl.ds(e0, _CH)], didx_v)
            pltpu.async_copy(h_hbm.at[gidx_v], rows_v, sem).wait()
            pltpu.sync_copy(rows_v, acc.at[didx_v], add=True)
            return carry

        lax.fori_loop(0, _NPW, body, 0)
        plsc.subcore_barrier()

        pltpu.sync_copy(acc.at[pl.ds(s * _RT, _RT)],
                        out_hbm.at[c, pl.ds(s * _RT, _RT)])

    return k(h_pad, eidx, zero_blk)


_BN = 400


def _tc_finalize(partials):
    def body(p_ref, o_ref):
        agg = p_ref[0] - p_ref[1]
        ss = jnp.sum(agg * agg, axis=1, keepdims=True)
        o_ref[...] = agg / (jnp.sqrt(ss) + 1e-7)

    return pl.pallas_call(
        body,
        grid=(_N // _BN,),
        in_specs=[pl.BlockSpec((_NC, _BN, _D), lambda i: (0, i, 0))],
        out_specs=pl.BlockSpec((_BN, _D), lambda i: (i, 0)),
        out_shape=jax.ShapeDtypeStruct((_N, _D), jnp.float32),
    )(partials)


def kernel(h, edge_index, r):
    eidx = jnp.concatenate(
        [edge_index.astype(jnp.int32),
         jnp.full((2, _EPAD - _E), _N, jnp.int32)], axis=1)
    h_pad = jnp.concatenate(
        [h, jnp.zeros((_HPAD, _D), jnp.float32)], axis=0)
    zero_blk = jnp.zeros((_RT, _D), jnp.float32)
    partials = _sc_two_sided_accumulate(h_pad, eidx, zero_blk)
    return _tc_finalize(partials)


# speedup vs baseline: 1.5607x; 1.0007x over previous
import functools

import jax
import jax.numpy as jnp
from jax import lax
from jax.experimental import pallas as pl
from jax.experimental.pallas import tpu as pltpu
from jax.experimental.pallas import tpu_sc as plsc

_N = 10000
_D = 128
_E = 320000
_NC = 2
_NS = 16
_CH = 128
_NPW = -(-_E // (_CH * _NS))
_EPAD = _NPW * _CH * _NS
_RT = 632
_NA = _RT * _NS
_HPAD = 8


def _sc_two_sided_accumulate(h_pad, eidx, zero_blk):
    mesh = plsc.VectorSubcoreMesh(core_axis_name="c", subcore_axis_name="s")

    @functools.partial(
        pl.kernel,
        out_type=jax.ShapeDtypeStruct((_NC, _NA, _D), jnp.float32),
        mesh=mesh,
        scratch_types=[
            pltpu.VMEM((_CH,), jnp.int32),
            pltpu.VMEM((_CH,), jnp.int32),
            pltpu.VMEM((_CH, _D), jnp.float32),
            pltpu.VMEM_SHARED((_NA, _D), jnp.float32),
            pltpu.SemaphoreType.DMA,
        ],
    )
    def k(h_hbm, e_hbm, z_hbm, out_hbm, gidx_v, didx_v, rows_v, acc, sem):
        c = lax.axis_index("c")
        s = lax.axis_index("s")

        pltpu.sync_copy(z_hbm, acc.at[pl.ds(s * _RT, _RT)])
        plsc.subcore_barrier()

        def body(j, carry):
            e0 = (s * _NPW + j) * _CH
            pltpu.sync_copy(e_hbm.at[c, pl.ds(e0, _CH)], gidx_v)
            pltpu.sync_copy(e_hbm.at[1, pl.ds(e0, _CH)], didx_v)
            pltpu.sync_copy(h_hbm.at[gidx_v], rows_v)
            pltpu.sync_copy(rows_v, acc.at[didx_v], add=True)
            return carry

        lax.fori_loop(0, _NPW, body, 0)
        plsc.subcore_barrier()

        pltpu.sync_copy(acc.at[pl.ds(s * _RT, _RT)],
                        out_hbm.at[c, pl.ds(s * _RT, _RT)])

    return k(h_pad, eidx, zero_blk)


_BN = 400


def _tc_finalize(partials):
    def body(p_ref, o_ref):
        agg = p_ref[0] - p_ref[1]
        ss = jnp.sum(agg * agg, axis=1, keepdims=True)
        o_ref[...] = agg / (jnp.sqrt(ss) + 1e-7)

    return pl.pallas_call(
        body,
        grid=(_N // _BN,),
        in_specs=[pl.BlockSpec((_NC, _BN, _D), lambda i: (0, i, 0))],
        out_specs=pl.BlockSpec((_BN, _D), lambda i: (i, 0)),
        out_shape=jax.ShapeDtypeStruct((_N, _D), jnp.float32),
    )(partials)


def kernel(h, edge_index, r):
    eidx = jnp.concatenate(
        [edge_index.astype(jnp.int32),
         jnp.full((2, _EPAD - _E), _N, jnp.int32)], axis=1)
    h_pad = jnp.concatenate(
        [h, jnp.zeros((_HPAD, _D), jnp.float32)], axis=0)
    zero_blk = jnp.zeros((_RT, _D), jnp.float32)
    partials = _sc_two_sided_accumulate(h_pad, eidx, zero_blk)
    return _tc_finalize(partials)
